# baseline (speedup 1.0000x reference)
import functools

import jax
import jax.numpy as jnp
from jax import lax
from jax.experimental import pallas as pl
from jax.experimental.pallas import tpu as pltpu
from jax.experimental.pallas import tpu_sc as plsc

NC = 2
NS = 16
NW = NC * NS
KE = 128
LANES = 16


def _node_pad(n):
  unit = NS * KE
  return ((n + 1 + unit - 1) // unit) * unit


def _edge_pad(e):
  unit = NW * KE * 8
  return ((e + unit - 1) // unit) * unit


def _sc_hist(dst_h, np_, ept):
  blocks = ept // KE
  rows_pt = np_ // NS
  copies = rows_pt // KE
  mesh = plsc.VectorSubcoreMesh(core_axis_name="c", subcore_axis_name="s")

  @functools.partial(
      pl.kernel,
      out_type=jax.ShapeDtypeStruct((NC, np_, 128), jnp.float32),
      mesh=mesh,
      scratch_types=[
          pltpu.VMEM((blocks, KE), jnp.int32),
          pltpu.VMEM((KE, 128), jnp.float32),
          pltpu.VMEM_SHARED((np_, 128), jnp.float32),
      ],
  )
  def hist(dst_hbm, out_hbm, didx_v, buf_v, hist_sh):
    c = lax.axis_index("c")
    s = lax.axis_index("s")
    wid = c * NS + s
    row_base = s * rows_pt

    def zero_chunk(t, _):
      buf_v[t // 8, pl.ds((t % 8) * LANES, LANES)] = jnp.zeros(
          (LANES,), jnp.float32)
      return 0
    lax.fori_loop(0, KE * 8, zero_chunk, 0)

    def zcopy(j, _):
      pltpu.sync_copy(buf_v, hist_sh.at[pl.ds(row_base + j * KE, KE)])
      return 0
    lax.fori_loop(0, copies, zcopy, 0)

    def one_chunk(t, _):
      buf_v[t // 8, pl.ds((t % 8) * LANES, LANES)] = jnp.ones(
          (LANES,), jnp.float32)
      return 0
    lax.fori_loop(0, KE * 8, one_chunk, 0)

    pltpu.sync_copy(dst_hbm.at[pl.ds(wid * blocks, blocks)], didx_v)
    plsc.subcore_barrier()

    def edge_block(i, _):
      pltpu.sync_copy(buf_v, hist_sh.at[didx_v.at[i]], add=True)
      return 0
    lax.fori_loop(0, blocks, edge_block, 0)

    plsc.subcore_barrier()
    pltpu.sync_copy(hist_sh.at[pl.ds(row_base, rows_pt)],
                    out_hbm.at[c, pl.ds(row_base, rows_pt)])

  return hist(dst_h)


NBUF = 4
KEA = 64


def _sc_aggregate(src_a, dst_a, y, np_, b0t, b1t, phases, bph):
  rows_pt = np_ // NS
  copies = rows_pt // KEA
  mesh = plsc.VectorSubcoreMesh(core_axis_name="c", subcore_axis_name="s")

  @functools.partial(
      pl.kernel,
      out_type=jax.ShapeDtypeStruct((NC, np_, 128), jnp.float32),
      mesh=mesh,
      scratch_types=[
          pltpu.VMEM((bph, KEA), jnp.int32),
          pltpu.VMEM((bph, KEA), jnp.int32),
          pltpu.VMEM((NBUF, KEA, 128), jnp.float32),
          pltpu.VMEM_SHARED((np_, 128), jnp.float32),
          pltpu.SemaphoreType.DMA((NBUF,)),
      ],
  )
  def agg(src_hbm, dst_hbm, y_hbm, out_hbm, sidx_v, didx_v, rows_v, acc_sh,
          sem):
    c = lax.axis_index("c")
    s = lax.axis_index("s")
    my_blocks = jnp.where(c == 0, b0t, b1t)
    my_base = jnp.where(c == 0, s * b0t, NS * b0t + s * b1t)
    row_base = s * rows_pt

    def zero_chunk(t, _):
      rows_v[0, t // 8, pl.ds((t % 8) * LANES, LANES)] = jnp.zeros(
          (LANES,), jnp.float32)
      return 0
    lax.fori_loop(0, KEA * 8, zero_chunk, 0)

    def zcopy(j, _):
      pltpu.sync_copy(rows_v.at[0], acc_sh.at[pl.ds(row_base + j * KEA, KEA)])
      return 0
    lax.fori_loop(0, copies, zcopy, 0)

    plsc.subcore_barrier()

    for p in range(phases):
      off = p * bph
      nblk = jnp.clip(my_blocks - off, 0, bph)

      @pl.when(nblk > 0)
      def _phase():
        pltpu.sync_copy(src_hbm.at[pl.ds(my_base + off, bph)], sidx_v)
        pltpu.sync_copy(dst_hbm.at[pl.ds(my_base + off, bph)], didx_v)
        for k in range(NBUF - 1):
          @pl.when(k < nblk)
          def _prime():
            pltpu.async_copy(y_hbm.at[sidx_v.at[k]], rows_v.at[k], sem.at[k])

        def edge_block(i, _):
          nxt = i + NBUF - 1

          @pl.when(nxt < nblk)
          def _prefetch():
            pltpu.async_copy(y_hbm.at[sidx_v.at[nxt]],
                             rows_v.at[nxt % NBUF], sem.at[nxt % NBUF])

          b = i % NBUF
          pltpu.make_async_copy(y_hbm.at[sidx_v.at[i]], rows_v.at[b],
                                sem.at[b]).wait()
          pltpu.sync_copy(rows_v.at[b], acc_sh.at[didx_v.at[i]], add=True)
          return 0
        lax.fori_loop(0, nblk, edge_block, 0)

    plsc.subcore_barrier()
    pltpu.sync_copy(acc_sh.at[pl.ds(row_base, rows_pt)],
                    out_hbm.at[c, pl.ds(row_base, rows_pt)])

  return agg(src_a, dst_a, y)


def _dinv_from_hist(h_ref):
  deg = h_ref[0, :, 0:1] + h_ref[1, :, 0:1] + 1.0
  return lax.rsqrt(deg)


def _tc_first(x_p, w1, hcat, np_, rb):
  def body(x_ref, w_ref, h_ref, y_ref):
    dinv = _dinv_from_hist(h_ref)
    xw = jnp.dot(x_ref[...], w_ref[...],
                 preferred_element_type=jnp.float32,
                 precision=lax.Precision.HIGHEST)
    y_ref[...] = xw * dinv

  return pl.pallas_call(
      body,
      grid=(np_ // rb,),
      in_specs=[
          pl.BlockSpec((rb, 128), lambda i: (i, 0)),
          pl.BlockSpec((128, 128), lambda i: (0, 0)),
          pl.BlockSpec((NC, rb, 128), lambda i: (0, i, 0)),
      ],
      out_specs=pl.BlockSpec((rb, 128), lambda i: (i, 0)),
      out_shape=jax.ShapeDtypeStruct((np_, 128), jnp.float32),
  )(x_p, w1, hcat)


def _tc_mid(acc, y1, b1r, w2, hcat, np_, rb):
  def body(a_ref, y_ref, b_ref, w_ref, h_ref, o_ref):
    dinv = _dinv_from_hist(h_ref)
    h = (a_ref[0] + a_ref[1] + y_ref[...]) * dinv + b_ref[...]
    hw = jnp.dot(h, w_ref[...],
                 preferred_element_type=jnp.float32,
                 precision=lax.Precision.HIGHEST)
    o_ref[...] = hw * dinv

  return pl.pallas_call(
      body,
      grid=(np_ // rb,),
      in_specs=[
          pl.BlockSpec((NC, rb, 128), lambda i: (0, i, 0)),
          pl.BlockSpec((rb, 128), lambda i: (i, 0)),
          pl.BlockSpec((1, 128), lambda i: (0, 0)),
          pl.BlockSpec((128, 128), lambda i: (0, 0)),
          pl.BlockSpec((NC, rb, 128), lambda i: (0, i, 0)),
      ],
      out_specs=pl.BlockSpec((rb, 128), lambda i: (i, 0)),
      out_shape=jax.ShapeDtypeStruct((np_, 128), jnp.float32),
  )(acc, y1, b1r, w2, hcat)


def _tc_last(acc, y2, b2r, hcat, np_, rb):
  def body(a_ref, y_ref, b_ref, h_ref, o_ref):
    dinv = _dinv_from_hist(h_ref)
    o_ref[...] = (a_ref[0] + a_ref[1] + y_ref[...]) * dinv + b_ref[...]

  return pl.pallas_call(
      body,
      grid=(np_ // rb,),
      in_specs=[
          pl.BlockSpec((NC, rb, 128), lambda i: (0, i, 0)),
          pl.BlockSpec((rb, 128), lambda i: (i, 0)),
          pl.BlockSpec((1, 128), lambda i: (0, 0)),
          pl.BlockSpec((NC, rb, 128), lambda i: (0, i, 0)),
      ],
      out_specs=pl.BlockSpec((rb, 128), lambda i: (i, 0)),
      out_shape=jax.ShapeDtypeStruct((np_, 128), jnp.float32),
  )(acc, y2, b2r, hcat)


def kernel(x, edge_index, W1, b1, W2, b2):
  n, d = x.shape
  e = edge_index.shape[1]
  np_ = _node_pad(n)
  e_pad = _edge_pad(e)
  ept = e_pad // NW
  rb = 1280 if np_ % 1280 == 0 else NS * KE

  blocks = ept // KE
  tblk_a = e_pad // KEA
  tpt = tblk_a // NS
  b0t = max(8, min(tpt, 8 * round(tpt * 0.73 / 8)))
  b1t = tpt - b0t
  phases = -(-max(b0t, b1t) // 48)
  bph = 8 * (-(-max(b0t, b1t) // (phases * 8)))

  src = edge_index[0].astype(jnp.int32)
  dst = edge_index[1].astype(jnp.int32)
  pad = e_pad - e + bph * KEA
  src = jnp.concatenate([src, jnp.zeros((pad,), jnp.int32)])
  dst = jnp.concatenate([dst, jnp.full((pad,), n, jnp.int32)])
  dst_h = dst[:e_pad].reshape(-1, KE)
  src_a = src.reshape(-1, KEA)
  dst_a = dst.reshape(-1, KEA)

  x_p = jnp.pad(x, ((0, np_ - n), (0, 0)))
  b1r = b1.reshape(1, d)
  b2r = b2.reshape(1, d)

  hcat = _sc_hist(dst_h, np_, ept)

  y1 = _tc_first(x_p, W1, hcat, np_, rb)
  acc1 = _sc_aggregate(src_a, dst_a, y1, np_, b0t, b1t, phases, bph)
  y2 = _tc_mid(acc1, y1, b1r, W2, hcat, np_, rb)
  acc2 = _sc_aggregate(src_a, dst_a, y2, np_, b0t, b1t, phases, bph)
  out = _tc_last(acc2, y2, b2r, hcat, np_, rb)
  return out[:n]


# speedup vs baseline: 1.6190x; 1.6190x over previous
import functools

import jax
import jax.numpy as jnp
from jax import lax
from jax.experimental import pallas as pl
from jax.experimental.pallas import tpu as pltpu
from jax.experimental.pallas import tpu_sc as plsc

NC = 2
NS = 16
NW = NC * NS
KE = 128
LANES = 16


def _node_pad(n):
  unit = NS * KE
  return ((n + 1 + unit - 1) // unit) * unit


def _edge_pad(e):
  unit = NW * KE
  return ((e + unit - 1) // unit) * unit


def _sc_hist(esd, np_, ept):
  blocks = ept // KE
  rows_pt = np_ // NS
  copies = rows_pt // KE
  mesh = plsc.VectorSubcoreMesh(core_axis_name="c", subcore_axis_name="s")

  @functools.partial(
      pl.kernel,
      out_type=jax.ShapeDtypeStruct((NC, np_, 128), jnp.float32),
      mesh=mesh,
      scratch_types=[
          pltpu.VMEM((blocks, 2, KE), jnp.int32),
          pltpu.VMEM((KE, 128), jnp.float32),
          pltpu.VMEM_SHARED((np_, 128), jnp.float32),
      ],
  )
  def hist(esd_hbm, out_hbm, eidx_v, buf_v, hist_sh):
    c = lax.axis_index("c")
    s = lax.axis_index("s")
    wid = c * NS + s
    row_base = s * rows_pt

    def zero_chunk(t, _):
      buf_v[t // 8, pl.ds((t % 8) * LANES, LANES)] = jnp.zeros(
          (LANES,), jnp.float32)
      return 0
    lax.fori_loop(0, KE * 8, zero_chunk, 0)

    def zcopy(j, _):
      pltpu.sync_copy(buf_v, hist_sh.at[pl.ds(row_base + j * KE, KE)])
      return 0
    lax.fori_loop(0, copies, zcopy, 0)

    def one_chunk(t, _):
      buf_v[t // 8, pl.ds((t % 8) * LANES, LANES)] = jnp.ones(
          (LANES,), jnp.float32)
      return 0
    lax.fori_loop(0, KE * 8, one_chunk, 0)

    pltpu.sync_copy(esd_hbm.at[pl.ds(wid * blocks, blocks)], eidx_v)
    plsc.subcore_barrier()

    def edge_block(i, _):
      pltpu.sync_copy(buf_v, hist_sh.at[eidx_v.at[i, 1]], add=True)
      return 0
    lax.fori_loop(0, blocks, edge_block, 0)

    plsc.subcore_barrier()
    pltpu.sync_copy(hist_sh.at[pl.ds(row_base, rows_pt)],
                    out_hbm.at[c, pl.ds(row_base, rows_pt)])

  return hist(esd)


NBUF = 4
KEA = 64


def _sc_aggregate(esd, y, np_, b0t, b1t, phases, bph):
  rows_pt = np_ // NS
  copies = rows_pt // KEA
  mesh = plsc.VectorSubcoreMesh(core_axis_name="c", subcore_axis_name="s")

  @functools.partial(
      pl.kernel,
      out_type=jax.ShapeDtypeStruct((NC, np_, 128), jnp.float32),
      mesh=mesh,
      scratch_types=[
          pltpu.VMEM((bph, 2, KEA), jnp.int32),
          pltpu.VMEM((NBUF, KEA, 128), jnp.float32),
          pltpu.VMEM_SHARED((np_, 128), jnp.float32),
          pltpu.SemaphoreType.DMA((NBUF,)),
      ],
  )
  def agg(esd_hbm, y_hbm, out_hbm, eidx_v, rows_v, acc_sh, sem):
    c = lax.axis_index("c")
    s = lax.axis_index("s")
    my_blocks = jnp.where(c == 0, b0t, b1t)
    my_base = jnp.where(c == 0, s * b0t, NS * b0t + s * b1t)
    row_base = s * rows_pt

    def zero_chunk(t, _):
      rows_v[0, t // 8, pl.ds((t % 8) * LANES, LANES)] = jnp.zeros(
          (LANES,), jnp.float32)
      return 0
    lax.fori_loop(0, KEA * 8, zero_chunk, 0)

    def zcopy(j, _):
      pltpu.sync_copy(rows_v.at[0], acc_sh.at[pl.ds(row_base + j * KEA, KEA)])
      return 0
    lax.fori_loop(0, copies, zcopy, 0)

    plsc.subcore_barrier()

    for p in range(phases):
      off = p * bph
      nblk = jnp.clip(my_blocks - off, 0, bph)

      @pl.when(nblk > 0)
      def _phase():
        pltpu.sync_copy(esd_hbm.at[pl.ds(my_base + off, bph)], eidx_v)
        for k in range(NBUF - 1):
          @pl.when(k < nblk)
          def _prime():
            pltpu.async_copy(y_hbm.at[eidx_v.at[k, 0]], rows_v.at[k],
                             sem.at[k])

        def edge_block(i, _):
          nxt = i + NBUF - 1

          @pl.when(nxt < nblk)
          def _prefetch():
            pltpu.async_copy(y_hbm.at[eidx_v.at[nxt, 0]],
                             rows_v.at[nxt % NBUF], sem.at[nxt % NBUF])

          b = i % NBUF
          pltpu.make_async_copy(y_hbm.at[eidx_v.at[i, 0]], rows_v.at[b],
                                sem.at[b]).wait()
          pltpu.sync_copy(rows_v.at[b], acc_sh.at[eidx_v.at[i, 1]], add=True)
          return 0
        lax.fori_loop(0, nblk, edge_block, 0)

    plsc.subcore_barrier()
    pltpu.sync_copy(acc_sh.at[pl.ds(row_base, rows_pt)],
                    out_hbm.at[c, pl.ds(row_base, rows_pt)])

  return agg(esd, y)


def _dinv_from_hist(h_ref):
  deg = h_ref[0, :, 0:1] + h_ref[1, :, 0:1] + 1.0
  return lax.rsqrt(deg)


def _tc_first(x_p, w1, hcat, np_, rb):
  def body(x_ref, w_ref, h_ref, y_ref):
    dinv = _dinv_from_hist(h_ref)
    xw = jnp.dot(x_ref[...], w_ref[...],
                 preferred_element_type=jnp.float32,
                 precision=lax.Precision.HIGHEST)
    y_ref[...] = xw * dinv

  return pl.pallas_call(
      body,
      grid=(np_ // rb,),
      in_specs=[
          pl.BlockSpec((rb, 128), lambda i: (i, 0)),
          pl.BlockSpec((128, 128), lambda i: (0, 0)),
          pl.BlockSpec((NC, rb, 128), lambda i: (0, i, 0)),
      ],
      out_specs=pl.BlockSpec((rb, 128), lambda i: (i, 0)),
      out_shape=jax.ShapeDtypeStruct((np_, 128), jnp.float32),
  )(x_p, w1, hcat)


def _tc_mid(acc, y1, b1r, w2, hcat, np_, rb):
  def body(a_ref, y_ref, b_ref, w_ref, h_ref, o_ref):
    dinv = _dinv_from_hist(h_ref)
    h = (a_ref[0] + a_ref[1] + y_ref[...]) * dinv + b_ref[...]
    hw = jnp.dot(h, w_ref[...],
                 preferred_element_type=jnp.float32,
                 precision=lax.Precision.HIGHEST)
    o_ref[...] = hw * dinv

  return pl.pallas_call(
      body,
      grid=(np_ // rb,),
      in_specs=[
          pl.BlockSpec((NC, rb, 128), lambda i: (0, i, 0)),
          pl.BlockSpec((rb, 128), lambda i: (i, 0)),
          pl.BlockSpec((1, 128), lambda i: (0, 0)),
          pl.BlockSpec((128, 128), lambda i: (0, 0)),
          pl.BlockSpec((NC, rb, 128), lambda i: (0, i, 0)),
      ],
      out_specs=pl.BlockSpec((rb, 128), lambda i: (i, 0)),
      out_shape=jax.ShapeDtypeStruct((np_, 128), jnp.float32),
  )(acc, y1, b1r, w2, hcat)


def _tc_last(acc, y2, b2r, hcat, np_, rb):
  def body(a_ref, y_ref, b_ref, h_ref, o_ref):
    dinv = _dinv_from_hist(h_ref)
    o_ref[...] = (a_ref[0] + a_ref[1] + y_ref[...]) * dinv + b_ref[...]

  return pl.pallas_call(
      body,
      grid=(np_ // rb,),
      in_specs=[
          pl.BlockSpec((NC, rb, 128), lambda i: (0, i, 0)),
          pl.BlockSpec((rb, 128), lambda i: (i, 0)),
          pl.BlockSpec((1, 128), lambda i: (0, 0)),
          pl.BlockSpec((NC, rb, 128), lambda i: (0, i, 0)),
      ],
      out_specs=pl.BlockSpec((rb, 128), lambda i: (i, 0)),
      out_shape=jax.ShapeDtypeStruct((np_, 128), jnp.float32),
  )(acc, y2, b2r, hcat)


def kernel(x, edge_index, W1, b1, W2, b2):
  n, d = x.shape
  e = edge_index.shape[1]
  np_ = _node_pad(n)
  e_pad = _edge_pad(e)
  ept = e_pad // NW
  rb = 1280 if np_ % 1280 == 0 else NS * KE

  blocks = ept // KE
  src = edge_index[0].astype(jnp.int32)
  dst = edge_index[1].astype(jnp.int32)
  if e_pad != e:
    pad = e_pad - e
    src = jnp.concatenate([src, jnp.zeros((pad,), jnp.int32)])
    dst = jnp.concatenate([dst, jnp.full((pad,), n, jnp.int32)])
  esd = jnp.stack([src.reshape(NW, blocks, KE),
                   dst.reshape(NW, blocks, KE)],
                  axis=2).reshape(NW * blocks, 2, KE)

  tblk_a = e_pad // KEA
  b0t = max(1, min(tblk_a // NS, round(tblk_a * 0.73 / NS)))
  b1t = tblk_a // NS - b0t
  phases = -(-max(b0t, b1t) // 53)
  bph = (max(b0t, b1t) + phases - 1) // phases
  esd_a = jnp.concatenate(
      [jnp.stack([src.reshape(NW, ept // KEA, KEA),
                  dst.reshape(NW, ept // KEA, KEA)],
                 axis=2).reshape(tblk_a, 2, KEA),
       jnp.zeros((bph, 2, KEA), jnp.int32)], axis=0)

  x_p = jnp.pad(x, ((0, np_ - n), (0, 0)))
  b1r = b1.reshape(1, d)
  b2r = b2.reshape(1, d)

  hcat = _sc_hist(esd, np_, ept)

  y1 = _tc_first(x_p, W1, hcat, np_, rb)
  acc1 = _sc_aggregate(esd_a, y1, np_, b0t, b1t, phases, bph)
  y2 = _tc_mid(acc1, y1, b1r, W2, hcat, np_, rb)
  acc2 = _sc_aggregate(esd_a, y2, np_, b0t, b1t, phases, bph)
  out = _tc_last(acc2, y2, b2r, hcat, np_, rb)
  return out[:n]
